# baseline (speedup 1.0000x reference)
import functools

import jax
import jax.numpy as jnp
import numpy as np
from jax.experimental import pallas as pl
from jax.experimental.pallas import tpu as pltpu

C = 8
G = 8
BR = 4
M = 256
DEPTH = 7
N_TREES = 4
PER = (BR ** (DEPTH + 1) - 1) // (BR - 1)
CG = C * G
LEVEL_SIZES = [BR ** l for l in range(DEPTH + 1)]
LEVEL_STARTS = [(BR ** l - 1) // (BR - 1) for l in range(DEPTH + 1)]
FL = [N_TREES * BR ** l for l in range(DEPTH + 1)]
CHUNK = 16384
TDEEP = 5


def _fll_kernel(*refs):
    x_refs = refs[3:3 + DEPTH + 1]
    out_ref = refs[3 + DEPTH + 1]
    sc = refs[3 + DEPTH + 2:]
    pr_refs = (None,) + sc[:DEPTH - 1]
    pn_refs = sc[DEPTH - 1:]
    f32 = jnp.float32

    la = jax.nn.softmax(refs[0][...], axis=1)
    bt = jax.nn.softmax(refs[1][...], axis=0)
    pi = jax.nn.softmax(refs[2][...], axis=1)
    bt_t = bt.T

    r64 = jax.lax.broadcasted_iota(jnp.int32, (CG, CG), 0)
    c64 = jax.lax.broadcasted_iota(jnp.int32, (CG, CG), 1)
    bmask = (r64 // C == c64 // C).astype(f32)
    bds = []
    for p in range(BR):
        tp = la[p * CG:(p + 1) * CG, :]
        bds.append(jnp.tile(tp, (1, C)) * bmask)
    bdst = [b.T for b in bds]

    s_mat = (jax.lax.broadcasted_iota(jnp.int32, (CG, G), 0) // C
             == jax.lax.broadcasted_iota(jnp.int32, (CG, G), 1)).astype(f32)
    sb_mat = (jax.lax.broadcasted_iota(jnp.int32, (G, CG), 0)
              == jax.lax.broadcasted_iota(jnp.int32, (G, CG), 1) // C).astype(f32)

    dot = functools.partial(jnp.dot, precision=jax.lax.Precision.DEFAULT,
                            preferred_element_type=f32)

    pmask = (jax.lax.broadcasted_iota(jnp.int32, (G, CG), 1) // C
             == jax.lax.broadcasted_iota(jnp.int32, (G, CG), 0)).astype(f32)
    p0 = dot(jnp.ones((N_TREES, G), f32), jnp.tile(pi, (1, C)) * pmask)

    for l in range(1, TDEEP):
        src = p0 if l == 1 else pr_refs[l - 1][...]
        for p in range(BR):
            pr_refs[l][p * FL[l - 1]:(p + 1) * FL[l - 1], :] = dot(src, bds[p])
    src_t = pr_refs[TDEEP - 1][...].T
    for l in range(TDEEP, DEPTH):
        for p in range(BR):
            pr_refs[l][:, p * FL[l - 1]:(p + 1) * FL[l - 1]] = dot(bdst[p], src_t)
        src_t = pr_refs[l][...]

    bt_t_bf = bt_t.astype(jnp.bfloat16)
    bt_bf = bt.astype(jnp.bfloat16)

    def emit_t(x_row):
        n = x_row.shape[1]
        oh_t = (x_row == jax.lax.broadcasted_iota(jnp.int32, (M, n), 0)
                ).astype(jnp.bfloat16)
        return dot(bt_t_bf, oh_t)

    acc_t = jnp.zeros((G, N_TREES), f32)
    for l in range(DEPTH, TDEEP - 1, -1):
        npa = FL[l - 1]
        ch = min(CHUNK, npa)
        tsel_t = (jax.lax.broadcasted_iota(jnp.int32, (ch, N_TREES), 0) % N_TREES
                  == jax.lax.broadcasted_iota(jnp.int32, (ch, N_TREES), 1)
                  ).astype(f32)
        for cs in range(0, npa, ch):
            prod = None
            for p in range(BR):
                b = emit_t(x_refs[l][:, p * npa + cs:p * npa + cs + ch])
                if l == DEPTH:
                    pr = dot(bdst[p], pr_refs[l - 1][:, cs:cs + ch])
                    m = pr * b
                    unnorm = m
                    qb = b
                else:
                    pr = pr_refs[l][:, p * npa + cs:p * npa + cs + ch]
                    m = pr * b
                    pp = pn_refs[l][:, p * npa + cs:p * npa + cs + ch]
                    mpp = m * pp
                    unnorm = m * mpp
                    qb = b * mpp
                nu = dot(sb_mat, unnorm)
                acc_t = acc_t + dot(jnp.log(nu), tsel_t)
                q = qb * dot(s_mat, 1.0 / nu)
                u = dot(bds[p], q)
                prod = u if prod is None else prod * u
            if l == TDEEP:
                pn_refs[l - 1][cs:cs + ch, :] = prod.T
            else:
                pn_refs[l - 1][:, cs:cs + ch] = prod

    def emit(x_row):
        n = x_row.shape[1]
        oh_t = (x_row == jax.lax.broadcasted_iota(jnp.int32, (M, n), 0)
                ).astype(jnp.bfloat16)
        return jax.lax.dot_general(oh_t, bt_bf, (((0,), (0,)), ((), ())),
                                   precision=jax.lax.Precision.DEFAULT,
                                   preferred_element_type=f32)

    acc = jnp.zeros((N_TREES, G), f32)
    for l in range(TDEEP - 1, 0, -1):
        npa = FL[l - 1]
        tsel = (jax.lax.broadcasted_iota(jnp.int32, (N_TREES, FL[l]), 1) % N_TREES
                == jax.lax.broadcasted_iota(jnp.int32, (N_TREES, FL[l]), 0)
                ).astype(f32)
        b = emit(x_refs[l][...])
        m = pr_refs[l][...] * b
        pp = pn_refs[l][...]
        mpp = m * pp
        unnorm = m * mpp
        nu = dot(unnorm, s_mat)
        acc = acc + dot(tsel, jnp.log(nu))
        q = (b * mpp) * dot(1.0 / nu, sb_mat)
        prod = None
        for p in range(BR):
            u = dot(q[p * npa:(p + 1) * npa, :], bdst[p])
            prod = u if prod is None else prod * u
        pn_refs[l - 1][...] = prod
    b = emit(x_refs[0][...])
    m = p0 * b
    nu = dot(m * m * pn_refs[0][...], s_mat)
    out_ref[...] = acc + acc_t.T + jnp.log(nu)


def _rev_perm_mat(k):
    idx = np.arange(BR ** k)
    rev = np.zeros_like(idx)
    t = idx.copy()
    for _ in range(k):
        rev = rev * BR + (t % BR)
        t //= BR
    mat = np.zeros((BR ** k, BR ** k), np.float32)
    mat[idx, rev] = 1.0
    return jnp.asarray(mat)


def kernel(lambda_A, lambda_B, lambda_Pi, x):
    la = jnp.transpose(lambda_A, (2, 3, 1, 0)).reshape(BR * CG, C)
    lb = jnp.transpose(lambda_B, (1, 2, 0)).reshape(M, CG)
    lpi = jnp.transpose(lambda_Pi, (1, 0))
    x2 = x.reshape(N_TREES, PER).astype(jnp.float32)
    pmats = {k: _rev_perm_mat(k) for k in range((DEPTH + 2) // 2 + 1)}
    hp = functools.partial(jnp.dot, precision=jax.lax.Precision.HIGHEST)
    xs = []
    for l in range(DEPTH + 1):
        xl = x2[:, LEVEL_STARTS[l]:LEVEL_STARTS[l] + LEVEL_SIZES[l]]
        d = l + 1
        a, b = d // 2, d - d // 2
        xm = xl.reshape(BR ** a, BR ** b)
        xm = hp(pmats[b], hp(pmats[a], xm).T)
        xs.append(xm.reshape(1, FL[l]).astype(jnp.int32))
    pr_scratch = [pltpu.VMEM((FL[l], CG), jnp.float32) if l < TDEEP
                  else pltpu.VMEM((CG, FL[l]), jnp.float32)
                  for l in range(1, DEPTH)]
    pn_scratch = [pltpu.VMEM((FL[l], CG), jnp.float32) if l < TDEEP
                  else pltpu.VMEM((CG, FL[l]), jnp.float32)
                  for l in range(DEPTH)]
    return pl.pallas_call(
        _fll_kernel,
        out_shape=jax.ShapeDtypeStruct((N_TREES, G), jnp.float32),
        scratch_shapes=pr_scratch + pn_scratch,
    )(la, lb, lpi, *xs)


# speedup vs baseline: 1.0083x; 1.0083x over previous
import functools

import jax
import jax.numpy as jnp
import numpy as np
from jax.experimental import pallas as pl
from jax.experimental.pallas import tpu as pltpu

C = 8
G = 8
BR = 4
M = 256
DEPTH = 7
N_TREES = 4
PER = (BR ** (DEPTH + 1) - 1) // (BR - 1)
CG = C * G
LEVEL_SIZES = [BR ** l for l in range(DEPTH + 1)]
LEVEL_STARTS = [(BR ** l - 1) // (BR - 1) for l in range(DEPTH + 1)]
FL = [N_TREES * BR ** l for l in range(DEPTH + 1)]
CHUNK = 8192
TDEEP = 5


def _fll_kernel(*refs):
    x_refs = refs[3:3 + DEPTH + 1]
    out_ref = refs[3 + DEPTH + 1]
    sc = refs[3 + DEPTH + 2:]
    pr_refs = (None,) + sc[:DEPTH - 1]
    pn_refs = sc[DEPTH - 1:]
    f32 = jnp.float32

    la = jax.nn.softmax(refs[0][...], axis=1)
    bt = jax.nn.softmax(refs[1][...], axis=0)
    pi = jax.nn.softmax(refs[2][...], axis=1)
    bt_t = bt.T

    r64 = jax.lax.broadcasted_iota(jnp.int32, (CG, CG), 0)
    c64 = jax.lax.broadcasted_iota(jnp.int32, (CG, CG), 1)
    bmask = (r64 // C == c64 // C).astype(f32)
    bds = []
    for p in range(BR):
        tp = la[p * CG:(p + 1) * CG, :]
        bds.append(jnp.tile(tp, (1, C)) * bmask)
    bdst = [b.T for b in bds]

    s_mat = (jax.lax.broadcasted_iota(jnp.int32, (CG, G), 0) // C
             == jax.lax.broadcasted_iota(jnp.int32, (CG, G), 1)).astype(f32)
    sb_mat = (jax.lax.broadcasted_iota(jnp.int32, (G, CG), 0)
              == jax.lax.broadcasted_iota(jnp.int32, (G, CG), 1) // C).astype(f32)

    dot = functools.partial(jnp.dot, precision=jax.lax.Precision.DEFAULT,
                            preferred_element_type=f32)

    pmask = (jax.lax.broadcasted_iota(jnp.int32, (G, CG), 1) // C
             == jax.lax.broadcasted_iota(jnp.int32, (G, CG), 0)).astype(f32)
    p0 = dot(jnp.ones((N_TREES, G), f32), jnp.tile(pi, (1, C)) * pmask)

    for l in range(1, TDEEP):
        src = p0 if l == 1 else pr_refs[l - 1][...]
        for p in range(BR):
            pr_refs[l][p * FL[l - 1]:(p + 1) * FL[l - 1], :] = dot(src, bds[p])
    src_t = pr_refs[TDEEP - 1][...].T
    for l in range(TDEEP, DEPTH):
        for p in range(BR):
            pr_refs[l][:, p * FL[l - 1]:(p + 1) * FL[l - 1]] = dot(bdst[p], src_t)
        src_t = pr_refs[l][...]

    bt_t_bf = bt_t.astype(jnp.bfloat16)
    bt_bf = bt.astype(jnp.bfloat16)

    def emit_t(x_row):
        n = x_row.shape[1]
        oh_t = (x_row == jax.lax.broadcasted_iota(jnp.int32, (M, n), 0)
                ).astype(jnp.bfloat16)
        return dot(bt_t_bf, oh_t)

    acc_t = jnp.zeros((G, N_TREES), f32)
    for l in range(DEPTH, TDEEP - 1, -1):
        npa = FL[l - 1]
        ch = min(CHUNK, npa)
        tsel_t = (jax.lax.broadcasted_iota(jnp.int32, (ch, N_TREES), 0) % N_TREES
                  == jax.lax.broadcasted_iota(jnp.int32, (ch, N_TREES), 1)
                  ).astype(f32)
        for cs in range(0, npa, ch):
            prod = None
            for p in range(BR):
                b = emit_t(x_refs[l][:, p * npa + cs:p * npa + cs + ch])
                if l == DEPTH:
                    pr = dot(bdst[p], pr_refs[l - 1][:, cs:cs + ch])
                    m = pr * b
                    unnorm = m
                    qb = b
                else:
                    pr = pr_refs[l][:, p * npa + cs:p * npa + cs + ch]
                    m = pr * b
                    pp = pn_refs[l][:, p * npa + cs:p * npa + cs + ch]
                    mpp = m * pp
                    unnorm = m * mpp
                    qb = b * mpp
                nu = dot(sb_mat, unnorm)
                acc_t = acc_t + dot(jnp.log(nu), tsel_t)
                q = qb * dot(s_mat, 1.0 / nu)
                u = dot(bds[p], q)
                prod = u if prod is None else prod * u
            if l == TDEEP:
                pn_refs[l - 1][cs:cs + ch, :] = prod.T
            else:
                pn_refs[l - 1][:, cs:cs + ch] = prod

    def emit(x_row):
        n = x_row.shape[1]
        oh_t = (x_row == jax.lax.broadcasted_iota(jnp.int32, (M, n), 0)
                ).astype(jnp.bfloat16)
        return jax.lax.dot_general(oh_t, bt_bf, (((0,), (0,)), ((), ())),
                                   precision=jax.lax.Precision.DEFAULT,
                                   preferred_element_type=f32)

    acc = jnp.zeros((N_TREES, G), f32)
    for l in range(TDEEP - 1, 0, -1):
        npa = FL[l - 1]
        tsel = (jax.lax.broadcasted_iota(jnp.int32, (N_TREES, FL[l]), 1) % N_TREES
                == jax.lax.broadcasted_iota(jnp.int32, (N_TREES, FL[l]), 0)
                ).astype(f32)
        b = emit(x_refs[l][...])
        m = pr_refs[l][...] * b
        pp = pn_refs[l][...]
        mpp = m * pp
        unnorm = m * mpp
        nu = dot(unnorm, s_mat)
        acc = acc + dot(tsel, jnp.log(nu))
        q = (b * mpp) * dot(1.0 / nu, sb_mat)
        prod = None
        for p in range(BR):
            u = dot(q[p * npa:(p + 1) * npa, :], bdst[p])
            prod = u if prod is None else prod * u
        pn_refs[l - 1][...] = prod
    b = emit(x_refs[0][...])
    m = p0 * b
    nu = dot(m * m * pn_refs[0][...], s_mat)
    out_ref[...] = acc + acc_t.T + jnp.log(nu)


def _rev_perm_mat(k):
    idx = np.arange(BR ** k)
    rev = np.zeros_like(idx)
    t = idx.copy()
    for _ in range(k):
        rev = rev * BR + (t % BR)
        t //= BR
    mat = np.zeros((BR ** k, BR ** k), np.float32)
    mat[idx, rev] = 1.0
    return jnp.asarray(mat)


def kernel(lambda_A, lambda_B, lambda_Pi, x):
    la = jnp.transpose(lambda_A, (2, 3, 1, 0)).reshape(BR * CG, C)
    lb = jnp.transpose(lambda_B, (1, 2, 0)).reshape(M, CG)
    lpi = jnp.transpose(lambda_Pi, (1, 0))
    x2 = x.reshape(N_TREES, PER).astype(jnp.float32)
    pmats = {k: _rev_perm_mat(k) for k in range((DEPTH + 2) // 2 + 1)}
    hp = functools.partial(jnp.dot, precision=jax.lax.Precision.HIGHEST)
    xs = []
    for l in range(DEPTH + 1):
        xl = x2[:, LEVEL_STARTS[l]:LEVEL_STARTS[l] + LEVEL_SIZES[l]]
        d = l + 1
        a, b = d // 2, d - d // 2
        xm = xl.reshape(BR ** a, BR ** b)
        xm = hp(pmats[b], hp(pmats[a], xm).T)
        xs.append(xm.reshape(1, FL[l]).astype(jnp.int32))
    pr_scratch = [pltpu.VMEM((FL[l], CG), jnp.float32) if l < TDEEP
                  else pltpu.VMEM((CG, FL[l]), jnp.float32)
                  for l in range(1, DEPTH)]
    pn_scratch = [pltpu.VMEM((FL[l], CG), jnp.float32) if l < TDEEP
                  else pltpu.VMEM((CG, FL[l]), jnp.float32)
                  for l in range(DEPTH)]
    return pl.pallas_call(
        _fll_kernel,
        out_shape=jax.ShapeDtypeStruct((N_TREES, G), jnp.float32),
        scratch_shapes=pr_scratch + pn_scratch,
    )(la, lb, lpi, *xs)
